# baseline (speedup 1.0000x reference)
import functools

import jax
import jax.numpy as jnp
from jax import lax
from jax.experimental import pallas as pl
from jax.experimental.pallas import tpu as pltpu
from jax.experimental.pallas import tpu_sc as plsc

N = 10000
NP = 10240
E_IN = 320000
ET = E_IN + N
NW = 32
CH1 = 64
CH2 = 128
STEPS2 = 2 * (-(-ET // (NW * CH2 * 2)))
PE = NW * CH2 * STEPS2
STEPS1 = PE // (NW * CH1)
IDX1 = STEPS1 + 2
IDX2 = STEPS2 + 2
ROWS_PER_SUB = NP // 16

_f32 = jnp.float32
_HI = lax.Precision.HIGHEST


def _tc1_body(x_ref, w_ref, a_ref, h_ref, t_ref):
    h = jnp.dot(x_ref[...], w_ref[...], precision=_HI,
                preferred_element_type=_f32)
    h_ref[...] = h
    t_ref[...] = jnp.dot(h, a_ref[...], precision=_HI,
                         preferred_element_type=_f32)


def _tc_stage1(xp, W1, A):
    B = 2048
    return pl.pallas_call(
        _tc1_body,
        grid=(NP // B,),
        in_specs=[
            pl.BlockSpec((B, 128), lambda i: (i, 0)),
            pl.BlockSpec((128, 128), lambda i: (0, 0)),
            pl.BlockSpec((128, 16), lambda i: (0, 0)),
        ],
        out_specs=[
            pl.BlockSpec((B, 128), lambda i: (i, 0)),
            pl.BlockSpec((B, 16), lambda i: (i, 0)),
        ],
        out_shape=[
            jax.ShapeDtypeStruct((NP, 128), _f32),
            jax.ShapeDtypeStruct((NP, 16), _f32),
        ],
    )(xp, W1, A)


def _tc2_body(op_ref, dp_ref, b1_ref, e_ref, m_ref, t2_ref):
    o = op_ref[0] + op_ref[1]
    den = dp_ref[0] + dp_ref[1]
    den128 = jnp.dot(den, e_ref[...], precision=_HI,
                     preferred_element_type=_f32)
    z = o / (den128 + 1e-16) + b1_ref[...]
    z = jnp.where(z > 0, z, jnp.exp(jnp.minimum(z, 0.0)) - 1.0)
    t2_ref[...] = jnp.dot(z, m_ref[...], precision=_HI,
                          preferred_element_type=_f32)


def _tc_stage2(outp, denp, b1, Eexp, M):
    B = 2048
    return pl.pallas_call(
        _tc2_body,
        grid=(NP // B,),
        in_specs=[
            pl.BlockSpec((2, B, 128), lambda i: (0, i, 0)),
            pl.BlockSpec((2, B, 16), lambda i: (0, i, 0)),
            pl.BlockSpec((1, 128), lambda i: (0, 0)),
            pl.BlockSpec((16, 128), lambda i: (0, 0)),
            pl.BlockSpec((128, 16), lambda i: (0, 0)),
        ],
        out_specs=pl.BlockSpec((B, 16), lambda i: (i, 0)),
        out_shape=jax.ShapeDtypeStruct((NP, 16), _f32),
    )(outp, denp, b1, Eexp, M)


def _tc3_body(ap_ref, b2_ref, o_ref):
    a = ap_ref[0] + ap_ref[1]
    r = a[:, 1:2] / (a[:, 0:1] + 1e-16) + b2_ref[0, 0]
    s = 1.0 / (1.0 + jnp.exp(-r))
    o_ref[...] = jnp.broadcast_to(s, o_ref.shape)


def _tc_stage3(accp, b2v):
    B = 2048
    return pl.pallas_call(
        _tc3_body,
        grid=(NP // B,),
        in_specs=[
            pl.BlockSpec((2, B, 16), lambda i: (0, i, 0)),
            pl.BlockSpec((1, 16), lambda i: (0, 0)),
        ],
        out_specs=pl.BlockSpec((B, 16), lambda i: (i, 0)),
        out_shape=jax.ShapeDtypeStruct((NP, 16), _f32),
    )(accp, b2v)


def _iota16():
    return lax.iota(jnp.int32, 16)


_DG_DNUMS = lax.GatherDimensionNumbers(
    offset_dims=(), collapsed_slice_dims=(0,), start_index_map=(0,))


def _dg(v, idx):
    return lax.gather(v, idx[:, None], _DG_DNUMS, slice_sizes=(1,),
                      mode=lax.GatherScatterMode.PROMISE_IN_BOUNDS)


_ZVEC = None


def _zero_vmem_rows(ref, nrows, width16):
    z = jnp.zeros((16,), _f32)

    @pl.loop(0, nrows)
    def _(i):
        for j in range(width16):
            ref[i, pl.ds(j * 16, 16)] = z


def _sc1_kernel(T_hbm, Tb_hbm, h_hbm, src_hbm, dst_hbm, c_hbm,
                out_hbm, den_hbm,
                out_sh, den_sh,
                srcv0, dstv0, dsc0, Ts0, Td0, hE0,
                srcv1, dstv1, dsc1, Ts1, Td1, hE1,
                cv, sg0, sg1, so0, so1, si0, si1):
    cid = lax.axis_index("c")
    sid = lax.axis_index("s")
    wid = cid * 16 + sid
    bufs = ((srcv0, dstv0, dsc0, Ts0, Td0, hE0, sg0, so0, si0),
            (srcv1, dstv1, dsc1, Ts1, Td1, hE1, sg1, so1, si1))

    _zero_vmem_rows(hE0, CH1, 8)
    _zero_vmem_rows(Ts0, CH1, 1)
    r0 = sid * ROWS_PER_SUB
    for k in range(ROWS_PER_SUB // CH1):
        pltpu.sync_copy(hE0, out_sh.at[pl.ds(r0 + k * CH1, CH1)])
        pltpu.sync_copy(Ts0, den_sh.at[pl.ds(r0 + k * CH1, CH1)])
    pltpu.sync_copy(c_hbm, cv)
    plsc.subcore_barrier()

    cvec = cv[0]
    base = wid * IDX1 * CH1

    def descs(b):
        srcv, dstv, dsc, Ts, Td, hE, sg, so, si = bufs[b]
        gath = (pltpu.make_async_copy(T_hbm.at[srcv], Ts, sg),
                pltpu.make_async_copy(Tb_hbm.at[dstv], Td, sg),
                pltpu.make_async_copy(h_hbm.at[srcv], hE, sg))
        scat = (pltpu.make_async_copy(Ts, den_sh.at[dsc], so),
                pltpu.make_async_copy(hE, out_sh.at[dsc], so))
        return gath, scat

    def idx_descs(gg, b):
        srcv, dstv = bufs[b][0], bufs[b][1]
        off = base + gg * CH1
        return (pltpu.make_async_copy(src_hbm.at[pl.ds(off, CH1)], srcv,
                                      bufs[b][8]),
                pltpu.make_async_copy(dst_hbm.at[pl.ds(off, CH1)], dstv,
                                      bufs[b][8]))

    for d in idx_descs(0, 0):
        d.start()
    for d in idx_descs(1, 1):
        d.start()
    for d in idx_descs(0, 0):
        d.wait()
    for d in descs(0)[0]:
        d.start()

    @pl.loop(0, STEPS1, step=2)
    def _(g):
        for b in (0, 1):
            gg = g + b
            nb = 1 - b
            gath, scat = descs(b)
            _, nscat = descs(nb)
            srcv, dstv, dsc, Ts, Td, hE, sg, so, si = bufs[b]

            @pl.when(gg >= 1)
            def _():
                for d in nscat:
                    d.wait()
            for d in idx_descs(gg + 1, nb):
                d.wait()
            for d in descs(nb)[0]:
                d.start()
            for d in gath:
                d.wait()
            for j in range(CH1 // 16):
                dsc[pl.ds(j * 16, 16)] = dstv[pl.ds(j * 16, 16)]
            for d in idx_descs(gg + 2, b):
                d.start()

            @pl.loop(0, CH1, unroll=4)
            def _(i):
                e = Ts[i] + Td[i]
                e = jnp.maximum(e, 0.2 * e)
                w = jnp.exp(e - cvec)
                Ts[i] = w
                for hd in range(8):
                    sp = _dg(w, jnp.full((16,), hd, jnp.int32))
                    hE[i, pl.ds(hd * 16, 16)] = hE[i, pl.ds(hd * 16, 16)] * sp

            for d in scat:
                d.start(add=True)

    for d in descs(1)[1]:
        d.wait()
    for d in descs(0)[0]:
        d.wait()
    for d in idx_descs(STEPS1 + 1, 1):
        d.wait()
    plsc.subcore_barrier()
    pltpu.sync_copy(out_sh.at[pl.ds(r0, ROWS_PER_SUB)],
                    out_hbm.at[cid, pl.ds(r0, ROWS_PER_SUB)])
    pltpu.sync_copy(den_sh.at[pl.ds(r0, ROWS_PER_SUB)],
                    den_hbm.at[cid, pl.ds(r0, ROWS_PER_SUB)])


_SC_PARAMS = pltpu.CompilerParams(use_tc_tiling_on_sc=False)


def _sc_layer1(T1, T1b, h1, src, dst, c16):
    mesh = plsc.VectorSubcoreMesh(core_axis_name="c", subcore_axis_name="s")
    buf = [pltpu.VMEM((CH1,), jnp.int32),
           pltpu.VMEM((CH1,), jnp.int32),
           pltpu.VMEM((CH1,), jnp.int32),
           pltpu.VMEM((CH1, 16), _f32),
           pltpu.VMEM((CH1, 16), _f32),
           pltpu.VMEM((CH1, 128), _f32)]
    return pl.kernel(
        _sc1_kernel,
        out_type=(jax.ShapeDtypeStruct((2, NP, 128), _f32),
                  jax.ShapeDtypeStruct((2, NP, 16), _f32)),
        mesh=mesh,
        scratch_types=[
            pltpu.VMEM_SHARED((NP, 128), _f32),
            pltpu.VMEM_SHARED((NP, 16), _f32),
            *buf, *buf,
            pltpu.VMEM((1, 16), _f32),
        ] + [pltpu.SemaphoreType.DMA] * 6,
        compiler_params=_SC_PARAMS,
    )(T1, T1b, h1, src, dst, c16)


def _sc2_kernel(T_hbm, Tb_hbm, src_hbm, dst_hbm, c_hbm,
                acc_hbm,
                acc_sh, T_sh, Tb_sh,
                srcv0, dstv0, dsc0, Ts0, Td0,
                srcv1, dstv1, dsc1, Ts1, Td1,
                cv, sg0, sg1, so0, so1, si0, si1):
    cid = lax.axis_index("c")
    sid = lax.axis_index("s")
    wid = cid * 16 + sid
    bufs = ((srcv0, dstv0, dsc0, Ts0, Td0, sg0, so0, si0),
            (srcv1, dstv1, dsc1, Ts1, Td1, sg1, so1, si1))

    _zero_vmem_rows(Ts0, CH2, 1)
    r0 = sid * ROWS_PER_SUB
    for k in range(ROWS_PER_SUB // CH2):
        pltpu.sync_copy(Ts0, acc_sh.at[pl.ds(r0 + k * CH2, CH2)])
    pltpu.sync_copy(T_hbm.at[pl.ds(r0, ROWS_PER_SUB)],
                    T_sh.at[pl.ds(r0, ROWS_PER_SUB)])
    pltpu.sync_copy(Tb_hbm.at[pl.ds(r0, ROWS_PER_SUB)],
                    Tb_sh.at[pl.ds(r0, ROWS_PER_SUB)])
    pltpu.sync_copy(c_hbm, cv)
    plsc.subcore_barrier()

    cvec = cv[0]
    ii = _iota16()
    zeros_i = jnp.full((16,), 0, jnp.int32)
    twos_i = jnp.full((16,), 2, jnp.int32)
    mask1 = ii == 1
    onehot0 = jnp.where(ii == 0, 1.0, 0.0).astype(_f32)
    base = wid * IDX2 * CH2

    def descs(b):
        srcv, dstv, dsc, Ts, Td, sg, so, si = bufs[b]
        gath = (pltpu.make_async_copy(T_sh.at[srcv], Ts, sg),
                pltpu.make_async_copy(Tb_sh.at[dstv], Td, sg))
        scat = (pltpu.make_async_copy(Ts, acc_sh.at[dsc], so),)
        return gath, scat

    def idx_descs(gg, b):
        srcv, dstv = bufs[b][0], bufs[b][1]
        off = base + gg * CH2
        return (pltpu.make_async_copy(src_hbm.at[pl.ds(off, CH2)], srcv,
                                      bufs[b][7]),
                pltpu.make_async_copy(dst_hbm.at[pl.ds(off, CH2)], dstv,
                                      bufs[b][7]))

    for d in idx_descs(0, 0):
        d.start()
    for d in idx_descs(1, 1):
        d.start()
    for d in idx_descs(0, 0):
        d.wait()
    for d in descs(0)[0]:
        d.start()

    @pl.loop(0, STEPS2, step=2)
    def _(g):
        for b in (0, 1):
            gg = g + b
            nb = 1 - b
            gath, scat = descs(b)
            _, nscat = descs(nb)
            srcv, dstv, dsc, Ts, Td, sg, so, si = bufs[b]

            @pl.when(gg >= 1)
            def _():
                for d in nscat:
                    d.wait()
            for d in idx_descs(gg + 1, nb):
                d.wait()
            for d in descs(nb)[0]:
                d.start()
            for d in gath:
                d.wait()
            for j in range(CH2 // 16):
                dsc[pl.ds(j * 16, 16)] = dstv[pl.ds(j * 16, 16)]
            for d in idx_descs(gg + 2, b):
                d.start()

            @pl.loop(0, CH2, unroll=4)
            def _(i):
                s = Ts[i]
                t = s + Td[i]
                t = jnp.maximum(t, 0.2 * t)
                u = jnp.exp(t - cvec)
                wspl = _dg(u, zeros_i)
                hspl = _dg(s, twos_i)
                Ts[i] = wspl * jnp.where(mask1, hspl, onehot0)

            for d in scat:
                d.start(add=True)

    for d in descs(1)[1]:
        d.wait()
    for d in descs(0)[0]:
        d.wait()
    for d in idx_descs(STEPS2 + 1, 1):
        d.wait()
    plsc.subcore_barrier()
    pltpu.sync_copy(acc_sh.at[pl.ds(r0, ROWS_PER_SUB)],
                    acc_hbm.at[cid, pl.ds(r0, ROWS_PER_SUB)])


def _sc_layer2(T2, T2b, src, dst, c2v):
    mesh = plsc.VectorSubcoreMesh(core_axis_name="c", subcore_axis_name="s")
    buf = [pltpu.VMEM((CH2,), jnp.int32),
           pltpu.VMEM((CH2,), jnp.int32),
           pltpu.VMEM((CH2,), jnp.int32),
           pltpu.VMEM((CH2, 16), _f32),
           pltpu.VMEM((CH2, 16), _f32)]
    return pl.kernel(
        _sc2_kernel,
        out_type=jax.ShapeDtypeStruct((2, NP, 16), _f32),
        mesh=mesh,
        scratch_types=[
            pltpu.VMEM_SHARED((NP, 16), _f32),
            pltpu.VMEM_SHARED((NP, 16), _f32),
            pltpu.VMEM_SHARED((NP, 16), _f32),
            *buf, *buf,
            pltpu.VMEM((1, 16), _f32),
        ] + [pltpu.SemaphoreType.DMA] * 6,
        compiler_params=_SC_PARAMS,
    )(T2, T2b, src, dst, c2v)


def kernel(x, edge_index, W1, a_src1, a_dst1, b1, W2, a_src2, a_dst2, b2):
    xp = jnp.zeros((NP, 128), _f32).at[:N].set(x)
    loop = jnp.arange(N, dtype=jnp.int32)
    pad = jnp.full((PE - ET,), N, jnp.int32)

    def _worker_idx(e, ch):
        e = jnp.concatenate([e.astype(jnp.int32), loop, pad]).reshape(NW, -1)
        tail = jnp.full((NW, 2 * ch), N, jnp.int32)
        return jnp.concatenate([e, tail], axis=1).reshape(-1)

    src1 = _worker_idx(edge_index[0], CH1)
    dst1 = _worker_idx(edge_index[1], CH1)
    src2 = _worker_idx(edge_index[0], CH2)
    dst2 = _worker_idx(edge_index[1], CH2)

    rows = jnp.arange(128)
    A = jnp.zeros((128, 16), _f32)
    A = A.at[rows, rows // 16].set(a_src1.reshape(128))
    A = A.at[rows, 8 + rows // 16].set(a_dst1.reshape(128))

    h1, T1 = _tc_stage1(xp, W1, A)
    T1b = jnp.concatenate([T1[:, 8:], T1[:, :8]], axis=1)
    c1 = T1[:, :8].max(0) + T1[:, 8:].max(0)
    c16 = jnp.concatenate([c1, jnp.full((8,), 1e30, _f32)]).reshape(1, 16)

    outp, denp = _sc_layer1(T1, T1b, h1, src1, dst1, c16)

    Eexp = jnp.zeros((16, 128), _f32).at[rows // 16, rows].set(1.0)
    v = jnp.concatenate([a_src2.reshape(1), a_dst2.reshape(1),
                         jnp.ones((1,), _f32), jnp.zeros((13,), _f32)])
    M = W2 * v[None, :]
    T2 = _tc_stage2(outp, denp, b1.reshape(1, 128), Eexp, M)
    T2b = T2[:, jnp.array([1, 0] + list(range(2, 16)))]
    c2 = T2[:, 0].max() + T2[:, 1].max()
    c2v = jnp.full((1, 16), 1e30, _f32).at[0, 0].set(c2)

    accp = _sc_layer2(T2, T2b, src2, dst2, c2v)

    res = _tc_stage3(accp, b2.reshape(1, 1) * jnp.ones((1, 16), _f32))
    return res[:N, 0:1]


# speedup vs baseline: 1.1779x; 1.1779x over previous
import jax
import jax.numpy as jnp
from jax import lax
from jax.experimental import pallas as pl
from jax.experimental.pallas import tpu as pltpu
from jax.experimental.pallas import tpu_sc as plsc

N = 10000
NP = 10240
E_IN = 320000
ET = E_IN + N
NW = 32
CH = 128
STEPS = -(-ET // (NW * CH))
PE = NW * CH * STEPS
ROWS_PER_SUB = NP // 16

_f32 = jnp.float32
_HI = lax.Precision.HIGHEST


def _tc1_body(x_ref, w_ref, a_ref, h_ref, t_ref):
    h = jnp.dot(x_ref[...], w_ref[...], precision=_HI,
                preferred_element_type=_f32)
    h_ref[...] = h
    t_ref[...] = jnp.dot(h, a_ref[...], precision=_HI,
                         preferred_element_type=_f32)


def _tc_stage1(xp, W1, A):
    B = 2048
    return pl.pallas_call(
        _tc1_body,
        grid=(NP // B,),
        in_specs=[
            pl.BlockSpec((B, 128), lambda i: (i, 0)),
            pl.BlockSpec((128, 128), lambda i: (0, 0)),
            pl.BlockSpec((128, 16), lambda i: (0, 0)),
        ],
        out_specs=[
            pl.BlockSpec((B, 128), lambda i: (i, 0)),
            pl.BlockSpec((B, 16), lambda i: (i, 0)),
        ],
        out_shape=[
            jax.ShapeDtypeStruct((NP, 128), _f32),
            jax.ShapeDtypeStruct((NP, 16), _f32),
        ],
    )(xp, W1, A)


def _tc2_body(op_ref, dp_ref, b1_ref, e_ref, m_ref, t2_ref):
    o = op_ref[0] + op_ref[1]
    den = dp_ref[0] + dp_ref[1]
    den128 = jnp.dot(den, e_ref[...], precision=_HI,
                     preferred_element_type=_f32)
    z = o / (den128 + 1e-16) + b1_ref[...]
    z = jnp.where(z > 0, z, jnp.exp(jnp.minimum(z, 0.0)) - 1.0)
    t2_ref[...] = jnp.dot(z, m_ref[...], precision=_HI,
                          preferred_element_type=_f32)


def _tc_stage2(outp, denp, b1, Eexp, M):
    B = 2048
    return pl.pallas_call(
        _tc2_body,
        grid=(NP // B,),
        in_specs=[
            pl.BlockSpec((2, B, 128), lambda i: (0, i, 0)),
            pl.BlockSpec((2, B, 16), lambda i: (0, i, 0)),
            pl.BlockSpec((1, 128), lambda i: (0, 0)),
            pl.BlockSpec((16, 128), lambda i: (0, 0)),
            pl.BlockSpec((128, 16), lambda i: (0, 0)),
        ],
        out_specs=pl.BlockSpec((B, 16), lambda i: (i, 0)),
        out_shape=jax.ShapeDtypeStruct((NP, 16), _f32),
    )(outp, denp, b1, Eexp, M)


def _tc3_body(ap_ref, b2_ref, o_ref):
    a = ap_ref[0] + ap_ref[1]
    r = a[:, 1:2] / (a[:, 0:1] + 1e-16) + b2_ref[0, 0]
    s = 1.0 / (1.0 + jnp.exp(-r))
    o_ref[...] = jnp.broadcast_to(s, o_ref.shape)


def _tc_stage3(accp, b2v):
    B = 2048
    return pl.pallas_call(
        _tc3_body,
        grid=(NP // B,),
        in_specs=[
            pl.BlockSpec((2, B, 16), lambda i: (0, i, 0)),
            pl.BlockSpec((1, 16), lambda i: (0, 0)),
        ],
        out_specs=pl.BlockSpec((B, 16), lambda i: (i, 0)),
        out_shape=jax.ShapeDtypeStruct((NP, 16), _f32),
    )(accp, b2v)


def _iota16():
    return lax.iota(jnp.int32, 16)


_DG_DNUMS = lax.GatherDimensionNumbers(
    offset_dims=(), collapsed_slice_dims=(0,), start_index_map=(0,))


def _dg(v, idx):
    return lax.gather(v, idx[:, None], _DG_DNUMS, slice_sizes=(1,),
                      mode=lax.GatherScatterMode.PROMISE_IN_BOUNDS)


def _zero_vmem_rows(ref, nrows, width16):
    z = jnp.zeros((16,), _f32)

    @pl.loop(0, nrows)
    def _(i):
        for j in range(width16):
            ref[i, pl.ds(j * 16, 16)] = z


def _sc1_kernel(T_hbm, h_hbm, src_hbm, dst_hbm, c_hbm,
                out_hbm, den_hbm,
                out_sh, den_sh,
                srcv, dstv, Tsrc, Tdst, wv, hE, cv):
    cid = lax.axis_index("c")
    sid = lax.axis_index("s")
    wid = cid * 16 + sid

    _zero_vmem_rows(hE, CH, 8)
    _zero_vmem_rows(wv, CH, 1)
    r0 = sid * ROWS_PER_SUB
    for k in range(ROWS_PER_SUB // CH):
        pltpu.sync_copy(hE, out_sh.at[pl.ds(r0 + k * CH, CH)])
        pltpu.sync_copy(wv, den_sh.at[pl.ds(r0 + k * CH, CH)])
    pltpu.sync_copy(c_hbm, cv)
    plsc.subcore_barrier()

    cvec = cv[0]
    shift8 = (_iota16() & 7) + 8
    base = wid * STEPS * CH

    @pl.loop(0, STEPS)
    def _(g):
        off = base + g * CH
        pltpu.sync_copy(src_hbm.at[pl.ds(off, CH)], srcv)
        pltpu.sync_copy(dst_hbm.at[pl.ds(off, CH)], dstv)
        pltpu.sync_copy(T_hbm.at[srcv], Tsrc)
        pltpu.sync_copy(T_hbm.at[dstv], Tdst)
        pltpu.sync_copy(h_hbm.at[srcv], hE)

        @pl.loop(0, CH)
        def _(i):
            e = Tsrc[i] + _dg(Tdst[i], shift8)
            e = jnp.maximum(e, 0.2 * e)
            wv[i] = jnp.exp(e - cvec)

        pltpu.sync_copy(wv, den_sh.at[dstv], add=True)

        @pl.loop(0, CH)
        def _(i):
            wrow = wv[i]
            for hd in range(8):
                sp = _dg(wrow, jnp.full((16,), hd, jnp.int32))
                hE[i, pl.ds(hd * 16, 16)] = hE[i, pl.ds(hd * 16, 16)] * sp

        pltpu.sync_copy(hE, out_sh.at[dstv], add=True)

    plsc.subcore_barrier()
    pltpu.sync_copy(out_sh.at[pl.ds(r0, ROWS_PER_SUB)],
                    out_hbm.at[cid, pl.ds(r0, ROWS_PER_SUB)])
    pltpu.sync_copy(den_sh.at[pl.ds(r0, ROWS_PER_SUB)],
                    den_hbm.at[cid, pl.ds(r0, ROWS_PER_SUB)])


_SC_PARAMS = pltpu.CompilerParams(use_tc_tiling_on_sc=False)


def _sc_layer1(T1, h1, src, dst, c16):
    mesh = plsc.VectorSubcoreMesh(core_axis_name="c", subcore_axis_name="s")
    return pl.kernel(
        _sc1_kernel,
        out_type=(jax.ShapeDtypeStruct((2, NP, 128), _f32),
                  jax.ShapeDtypeStruct((2, NP, 16), _f32)),
        mesh=mesh,
        scratch_types=[
            pltpu.VMEM_SHARED((NP, 128), _f32),
            pltpu.VMEM_SHARED((NP, 16), _f32),
            pltpu.VMEM((CH,), jnp.int32),
            pltpu.VMEM((CH,), jnp.int32),
            pltpu.VMEM((CH, 16), _f32),
            pltpu.VMEM((CH, 16), _f32),
            pltpu.VMEM((CH, 16), _f32),
            pltpu.VMEM((CH, 128), _f32),
            pltpu.VMEM((1, 16), _f32),
        ],
        compiler_params=_SC_PARAMS,
    )(T1, h1, src, dst, c16)


def _sc2_kernel(T_hbm, src_hbm, dst_hbm, c_hbm,
                acc_hbm,
                acc_sh, T_sh,
                srcv, dstv, Tsrc, Tdst, rowv, cv):
    cid = lax.axis_index("c")
    sid = lax.axis_index("s")
    wid = cid * 16 + sid

    _zero_vmem_rows(rowv, CH, 1)
    r0 = sid * ROWS_PER_SUB
    for k in range(ROWS_PER_SUB // CH):
        pltpu.sync_copy(rowv, acc_sh.at[pl.ds(r0 + k * CH, CH)])
    pltpu.sync_copy(T_hbm.at[pl.ds(r0, ROWS_PER_SUB)],
                    T_sh.at[pl.ds(r0, ROWS_PER_SUB)])
    pltpu.sync_copy(c_hbm, cv)
    plsc.subcore_barrier()

    cvec = cv[0]
    ii = _iota16()
    ones_i = jnp.full((16,), 1, jnp.int32)
    zeros_i = jnp.full((16,), 0, jnp.int32)
    twos_i = jnp.full((16,), 2, jnp.int32)
    onehot0 = jnp.where(ii == 0, 1.0, 0.0).astype(_f32)
    base = wid * STEPS * CH

    @pl.loop(0, STEPS)
    def _(g):
        off = base + g * CH
        pltpu.sync_copy(src_hbm.at[pl.ds(off, CH)], srcv)
        pltpu.sync_copy(dst_hbm.at[pl.ds(off, CH)], dstv)
        pltpu.sync_copy(T_sh.at[srcv], Tsrc)
        pltpu.sync_copy(T_sh.at[dstv], Tdst)

        @pl.loop(0, CH)
        def _(i):
            s = Tsrc[i]
            t = s + _dg(Tdst[i], ones_i)
            t = jnp.maximum(t, 0.2 * t)
            u = jnp.exp(t - cvec)
            wspl = _dg(u, zeros_i)
            hspl = _dg(s, twos_i)
            rowv[i] = wspl * jnp.where(ii == 1, hspl, onehot0)

        pltpu.sync_copy(rowv, acc_sh.at[dstv], add=True)

    plsc.subcore_barrier()
    pltpu.sync_copy(acc_sh.at[pl.ds(r0, ROWS_PER_SUB)],
                    acc_hbm.at[cid, pl.ds(r0, ROWS_PER_SUB)])


def _sc_layer2(T2, src, dst, c2v):
    mesh = plsc.VectorSubcoreMesh(core_axis_name="c", subcore_axis_name="s")
    return pl.kernel(
        _sc2_kernel,
        out_type=jax.ShapeDtypeStruct((2, NP, 16), _f32),
        mesh=mesh,
        scratch_types=[
            pltpu.VMEM_SHARED((NP, 16), _f32),
            pltpu.VMEM_SHARED((NP, 16), _f32),
            pltpu.VMEM((CH,), jnp.int32),
            pltpu.VMEM((CH,), jnp.int32),
            pltpu.VMEM((CH, 16), _f32),
            pltpu.VMEM((CH, 16), _f32),
            pltpu.VMEM((CH, 16), _f32),
            pltpu.VMEM((1, 16), _f32),
        ],
        compiler_params=_SC_PARAMS,
    )(T2, src, dst, c2v)


def kernel(x, edge_index, W1, a_src1, a_dst1, b1, W2, a_src2, a_dst2, b2):
    xp = jnp.zeros((NP, 128), _f32).at[:N].set(x)
    loop = jnp.arange(N, dtype=jnp.int32)
    src = jnp.concatenate([edge_index[0].astype(jnp.int32), loop])
    dst = jnp.concatenate([edge_index[1].astype(jnp.int32), loop])
    pad = jnp.full((PE - ET,), N, jnp.int32)
    src = jnp.concatenate([src, pad])
    dst = jnp.concatenate([dst, pad])

    rows = jnp.arange(128)
    A = jnp.zeros((128, 16), _f32)
    A = A.at[rows, rows // 16].set(a_src1.reshape(128))
    A = A.at[rows, 8 + rows // 16].set(a_dst1.reshape(128))

    h1, T1 = _tc_stage1(xp, W1, A)
    c1 = T1[:, :8].max(0) + T1[:, 8:].max(0)
    c16 = jnp.concatenate([c1, jnp.full((8,), 1e30, _f32)]).reshape(1, 16)

    outp, denp = _sc_layer1(T1, h1, src, dst, c16)

    Eexp = jnp.zeros((16, 128), _f32).at[rows // 16, rows].set(1.0)
    v = jnp.concatenate([a_src2.reshape(1), a_dst2.reshape(1),
                         jnp.ones((1,), _f32), jnp.zeros((13,), _f32)])
    M = W2 * v[None, :]
    T2 = _tc_stage2(outp, denp, b1.reshape(1, 128), Eexp, M)
    c2 = T2[:, 0].max() + T2[:, 1].max()
    c2v = jnp.full((1, 16), 1e30, _f32).at[0, 0].set(c2)

    accp = _sc_layer2(T2, src, dst, c2v)

    res = _tc_stage3(accp, b2.reshape(1, 1) * jnp.ones((1, 16), _f32))
    return res[:N, 0:1]
